# baseline (speedup 1.0000x reference)
import functools

import jax
import jax.numpy as jnp
from jax import lax
from jax.experimental import pallas as pl
from jax.experimental.pallas import tpu as pltpu
from jax.experimental.pallas import tpu_sc as plsc

D = 1024
L = 16
NVEC = D // L
NC, NS = 2, 16
NW = NC * NS
EPS = 1e-5
CH = 32


JBLK = 16
NBLK = NVEC // JBLK


def _compute_chunk(rows_v, typ_v, nw_v, sums_v, yv_v):
    zero = jnp.zeros((L,), jnp.float32)

    for j0 in range(NBLK):
        t = [typ_v[pl.ds((j0 * JBLK + k) * L, L)] for k in range(JBLK)]

        @plsc.parallel_loop(0, CH, unroll=2)
        def row_p1(rr, j0=j0, t=t):
            accs = [zero, zero, zero, zero]
            for k in range(JBLK):
                sl = pl.ds((j0 * JBLK + k) * L, L)
                v = rows_v[rr, sl] + t[k]
                rows_v[rr, sl] = v
                accs[k % 4] = accs[k % 4] + v * v
            acc = (accs[0] + accs[1]) + (accs[2] + accs[3])
            if j0 == 0:
                sums_v[pl.ds(rr * L, L)] = acc
            else:
                plsc.addupdate(sums_v.at[pl.ds(rr * L, L)], acc)

    for bb in range(CH // L):
        col = lax.iota(jnp.int32, L) * L + (bb * L * L)

        def tr(cc, a, col=col):
            return a + plsc.load_gather(sums_v, [col + cc])

        acc2 = lax.fori_loop(0, L, tr, zero, unroll=True)
        var = acc2 * (1.0 / D) + EPS
        i = lax.bitcast_convert_type(var, jnp.int32)
        i = jnp.full((L,), 0x5F3759DF, jnp.int32) - \
            lax.shift_right_arithmetic(i, jnp.full((L,), 1, jnp.int32))
        y = lax.bitcast_convert_type(i, jnp.float32)
        h = 0.5 * var
        y = y * (1.5 - h * y * y)
        y = y * (1.5 - h * y * y)
        y = y * (1.5 - h * y * y)
        yv_v[pl.ds(bb * L, L)] = y

    for j0 in range(NBLK):
        w = [nw_v[pl.ds((j0 * JBLK + k) * L, L)] for k in range(JBLK)]

        @plsc.parallel_loop(0, CH, unroll=2)
        def row_p2(rr, j0=j0, w=w):
            s = plsc.load_gather(yv_v, [rr + jnp.zeros((L,), jnp.int32)])
            for k in range(JBLK):
                sl = pl.ds((j0 * JBLK + k) * L, L)
                rows_v[rr, sl] = rows_v[rr, sl] * s * w[k]


def _sc_body(ids_hbm, table_hbm, typ_hbm, nw_hbm, out_hbm,
             idx_v, rows0_v, rows1_v, typ_v, nw_v, sums_v, yv_v,
             sg0, sg1, ss0, ss1, tpw):
    wid = lax.axis_index("s") * NC + lax.axis_index("c")
    base = wid * tpw

    pltpu.sync_copy(ids_hbm.at[pl.ds(base, tpw)], idx_v)
    pltpu.sync_copy(typ_hbm, typ_v)
    pltpu.sync_copy(nw_hbm, nw_v)

    nch = tpw // CH
    npair = nch // 2

    def gather(c, buf, sem):
        pltpu.async_copy(table_hbm.at[idx_v.at[pl.ds(c * CH, CH)]], buf, sem)

    def wait_gather(buf, sem):
        pltpu.make_async_copy(
            table_hbm.at[idx_v.at[pl.ds(0, CH)]], buf, sem).wait()

    def scatter(c, buf, sem):
        pltpu.async_copy(buf, out_hbm.at[pl.ds(base + c * CH, CH)], sem)

    def wait_scatter(buf, sem):
        pltpu.make_async_copy(buf, out_hbm.at[pl.ds(base, CH)], sem).wait()

    gather(0, rows0_v, sg0)

    def body(g, carry):
        c0 = 2 * g
        gather(c0 + 1, rows1_v, sg1)
        wait_gather(rows0_v, sg0)
        _compute_chunk(rows0_v, typ_v, nw_v, sums_v, yv_v)
        scatter(c0, rows0_v, ss0)
        wait_gather(rows1_v, sg1)
        _compute_chunk(rows1_v, typ_v, nw_v, sums_v, yv_v)
        scatter(c0 + 1, rows1_v, ss1)
        wait_scatter(rows0_v, ss0)

        @pl.when(g < npair - 1)
        def _():
            gather(c0 + 2, rows0_v, sg0)

        wait_scatter(rows1_v, ss1)
        return carry

    lax.fori_loop(0, npair, body, 0)


def kernel(input_ids, token_type_ids, word_emb, type_emb, norm_w):
    del token_type_ids
    b, s = input_ids.shape
    n = b * s
    tpw = n // NW
    ids = input_ids.reshape(n)
    typ_row = type_emb.reshape(type_emb.shape[-1])

    mesh = plsc.VectorSubcoreMesh(
        core_axis_name="c", subcore_axis_name="s",
        num_cores=NC, num_subcores=NS)

    k = pl.kernel(
        functools.partial(_sc_body, tpw=tpw),
        out_type=jax.ShapeDtypeStruct((n, D), jnp.float32),
        mesh=mesh,
        compiler_params=pltpu.CompilerParams(needs_layout_passes=False),
        scratch_types=[
            pltpu.VMEM((tpw,), jnp.int32),
            pltpu.VMEM((CH, D), jnp.float32),
            pltpu.VMEM((CH, D), jnp.float32),
            pltpu.VMEM((D,), jnp.float32),
            pltpu.VMEM((D,), jnp.float32),
            pltpu.VMEM((CH * L,), jnp.float32),
            pltpu.VMEM((CH,), jnp.float32),
            pltpu.SemaphoreType.DMA,
            pltpu.SemaphoreType.DMA,
            pltpu.SemaphoreType.DMA,
            pltpu.SemaphoreType.DMA,
        ],
    )
    out = k(ids, word_emb, typ_row, norm_w)
    return out.reshape(b, s, D)


# speedup vs baseline: 1.0882x; 1.0882x over previous
import functools

import jax
import jax.numpy as jnp
from jax import lax
from jax.experimental import pallas as pl
from jax.experimental.pallas import tpu as pltpu
from jax.experimental.pallas import tpu_sc as plsc

D = 1024
L = 16
NVEC = D // L
NC, NS = 2, 16
NW = NC * NS
EPS = 1e-5
CH = 32


JBLK = 16
NBLK = NVEC // JBLK


def _compute_chunk(rows_v, typ_v, nw_v, sums_v, yv_v):
    zero = jnp.zeros((L,), jnp.float32)

    for j0 in range(NBLK):
        t = [typ_v[pl.ds((j0 * JBLK + k) * L, L)] for k in range(JBLK)]

        @plsc.parallel_loop(0, CH)
        def row_p1(rr, j0=j0, t=t):
            accs = [zero, zero, zero, zero]
            for k in range(JBLK):
                sl = pl.ds((j0 * JBLK + k) * L, L)
                v = rows_v[rr, sl] + t[k]
                rows_v[rr, sl] = v
                accs[k % 4] = accs[k % 4] + v * v
            acc = (accs[0] + accs[1]) + (accs[2] + accs[3])
            if j0 == 0:
                sums_v[pl.ds(rr * L, L)] = acc
            else:
                plsc.addupdate(sums_v.at[pl.ds(rr * L, L)], acc)

    for bb in range(CH // L):
        col = lax.iota(jnp.int32, L) * L + (bb * L * L)

        def tr(cc, a, col=col):
            return a + plsc.load_gather(sums_v, [col + cc])

        acc2 = lax.fori_loop(0, L, tr, zero, unroll=True)
        var = acc2 * (1.0 / D) + EPS
        i = lax.bitcast_convert_type(var, jnp.int32)
        i = jnp.full((L,), 0x5F3759DF, jnp.int32) - \
            lax.shift_right_arithmetic(i, jnp.full((L,), 1, jnp.int32))
        y = lax.bitcast_convert_type(i, jnp.float32)
        h = 0.5 * var
        y = y * (1.5 - h * y * y)
        y = y * (1.5 - h * y * y)
        y = y * (1.5 - h * y * y)
        yv_v[pl.ds(bb * L, L)] = y

    for j0 in range(NBLK):
        w = [nw_v[pl.ds((j0 * JBLK + k) * L, L)] for k in range(JBLK)]

        @plsc.parallel_loop(0, CH)
        def row_p2(rr, j0=j0, w=w):
            s = plsc.load_gather(yv_v, [rr + jnp.zeros((L,), jnp.int32)])
            for k in range(JBLK):
                sl = pl.ds((j0 * JBLK + k) * L, L)
                rows_v[rr, sl] = rows_v[rr, sl] * s * w[k]


def _sc_body(ids_hbm, table_hbm, typ_hbm, nw_hbm, out_hbm,
             idx_v, rows0_v, rows1_v, typ_v, nw_v, sums_v, yv_v,
             sg0, sg1, ss0, ss1, tpw):
    wid = lax.axis_index("s") * NC + lax.axis_index("c")
    base = wid * tpw

    pltpu.sync_copy(ids_hbm.at[pl.ds(base, tpw)], idx_v)
    pltpu.sync_copy(typ_hbm, typ_v)
    pltpu.sync_copy(nw_hbm, nw_v)

    nch = tpw // CH
    npair = nch // 2

    def gather(c, buf, sem):
        pltpu.async_copy(table_hbm.at[idx_v.at[pl.ds(c * CH, CH)]], buf, sem)

    def wait_gather(buf, sem):
        pltpu.make_async_copy(
            table_hbm.at[idx_v.at[pl.ds(0, CH)]], buf, sem).wait()

    def scatter(c, buf, sem):
        pltpu.async_copy(buf, out_hbm.at[pl.ds(base + c * CH, CH)], sem)

    def wait_scatter(buf, sem):
        pltpu.make_async_copy(buf, out_hbm.at[pl.ds(base, CH)], sem).wait()

    gather(0, rows0_v, sg0)

    def body(g, carry):
        c0 = 2 * g
        gather(c0 + 1, rows1_v, sg1)
        wait_gather(rows0_v, sg0)
        _compute_chunk(rows0_v, typ_v, nw_v, sums_v, yv_v)
        scatter(c0, rows0_v, ss0)
        wait_gather(rows1_v, sg1)
        _compute_chunk(rows1_v, typ_v, nw_v, sums_v, yv_v)
        scatter(c0 + 1, rows1_v, ss1)
        wait_scatter(rows0_v, ss0)

        @pl.when(g < npair - 1)
        def _():
            gather(c0 + 2, rows0_v, sg0)

        wait_scatter(rows1_v, ss1)
        return carry

    lax.fori_loop(0, npair, body, 0)


def kernel(input_ids, token_type_ids, word_emb, type_emb, norm_w):
    del token_type_ids
    b, s = input_ids.shape
    n = b * s
    tpw = n // NW
    ids = input_ids.reshape(n)
    typ_row = type_emb.reshape(type_emb.shape[-1])

    mesh = plsc.VectorSubcoreMesh(
        core_axis_name="c", subcore_axis_name="s",
        num_cores=NC, num_subcores=NS)

    k = pl.kernel(
        functools.partial(_sc_body, tpw=tpw),
        out_type=jax.ShapeDtypeStruct((n, D), jnp.float32),
        mesh=mesh,
        compiler_params=pltpu.CompilerParams(needs_layout_passes=False),
        scratch_types=[
            pltpu.VMEM((tpw,), jnp.int32),
            pltpu.VMEM((CH, D), jnp.float32),
            pltpu.VMEM((CH, D), jnp.float32),
            pltpu.VMEM((D,), jnp.float32),
            pltpu.VMEM((D,), jnp.float32),
            pltpu.VMEM((CH * L,), jnp.float32),
            pltpu.VMEM((CH,), jnp.float32),
            pltpu.SemaphoreType.DMA,
            pltpu.SemaphoreType.DMA,
            pltpu.SemaphoreType.DMA,
            pltpu.SemaphoreType.DMA,
        ],
    )
    out = k(ids, word_emb, typ_row, norm_w)
    return out.reshape(b, s, D)


# speedup vs baseline: 1.1916x; 1.0950x over previous
import functools

import jax
import jax.numpy as jnp
from jax import lax
from jax.experimental import pallas as pl
from jax.experimental.pallas import tpu as pltpu
from jax.experimental.pallas import tpu_sc as plsc

D = 1024
L = 16
NVEC = D // L
NC, NS = 2, 16
NW = NC * NS
EPS = 1e-5
CH = 32


JBLK = 16
NBLK = NVEC // JBLK


def _pass1_chunk(rows_v, typ_v, nw_v, sums_v, yv_v):
    zero = jnp.zeros((L,), jnp.float32)

    for j0 in range(NBLK):
        t = [typ_v[pl.ds((j0 * JBLK + k) * L, L)] for k in range(JBLK)]

        @plsc.parallel_loop(0, CH)
        def row_p1(rr, j0=j0, t=t):
            accs = [zero, zero, zero, zero]
            for k in range(JBLK):
                sl = pl.ds((j0 * JBLK + k) * L, L)
                v = rows_v[rr, sl] + t[k]
                rows_v[rr, sl] = v
                accs[k % 4] = accs[k % 4] + v * v
            acc = (accs[0] + accs[1]) + (accs[2] + accs[3])
            if j0 == 0:
                sums_v[pl.ds(rr * L, L)] = acc
            else:
                plsc.addupdate(sums_v.at[pl.ds(rr * L, L)], acc)

    for bb in range(CH // L):
        col = lax.iota(jnp.int32, L) * L + (bb * L * L)

        def tr(cc, a, col=col):
            return a + plsc.load_gather(sums_v, [col + cc])

        acc2 = lax.fori_loop(0, L, tr, zero, unroll=True)
        var = acc2 * (1.0 / D) + EPS
        i = lax.bitcast_convert_type(var, jnp.int32)
        i = jnp.full((L,), 0x5F3759DF, jnp.int32) - \
            lax.shift_right_arithmetic(i, jnp.full((L,), 1, jnp.int32))
        y = lax.bitcast_convert_type(i, jnp.float32)
        h = 0.5 * var
        y = y * (1.5 - h * y * y)
        y = y * (1.5 - h * y * y)
        y = y * (1.5 - h * y * y)
        yv_v[pl.ds(bb * L, L)] = y


def _pass2_chunk(rows_v, nw_v, yv_v):
    for j0 in range(NBLK):
        w = [nw_v[pl.ds((j0 * JBLK + k) * L, L)] for k in range(JBLK)]

        @plsc.parallel_loop(0, CH)
        def row_p2(rr, j0=j0, w=w):
            s = plsc.load_gather(yv_v, [rr + jnp.zeros((L,), jnp.int32)])
            for k in range(JBLK):
                sl = pl.ds((j0 * JBLK + k) * L, L)
                rows_v[rr, sl] = rows_v[rr, sl] * s * w[k]


def _sc_body(ids_hbm, table_hbm, typ_hbm, nw_hbm, out_hbm,
             idx_v, rows0_v, rows1_v, typ_v, nw_v, sums_v, yv_v,
             sg0, sg1, ss0, ss1, tpw):
    wid = lax.axis_index("s") * NC + lax.axis_index("c")
    base = wid * tpw

    nch = tpw // CH
    npair = nch // 2

    def gather(c, buf, sem):
        pltpu.async_copy(table_hbm.at[idx_v.at[pl.ds(c * CH, CH)]], buf, sem)

    def wait_gather(buf, sem):
        pltpu.make_async_copy(
            table_hbm.at[idx_v.at[pl.ds(0, CH)]], buf, sem).wait()

    def scatter(c, buf, sem):
        pltpu.async_copy(buf, out_hbm.at[pl.ds(base + c * CH, CH)], sem)

    def wait_scatter(buf, sem):
        pltpu.make_async_copy(buf, out_hbm.at[pl.ds(base, CH)], sem).wait()

    pltpu.sync_copy(ids_hbm.at[pl.ds(base, CH)], idx_v.at[pl.ds(0, CH)])
    gather(0, rows0_v, sg0)
    pltpu.sync_copy(ids_hbm.at[pl.ds(base + CH, tpw - CH)],
                    idx_v.at[pl.ds(CH, tpw - CH)])
    pltpu.sync_copy(typ_hbm, typ_v)
    pltpu.sync_copy(nw_hbm, nw_v)

    def body(g, carry):
        c0 = 2 * g
        wait_gather(rows0_v, sg0)
        _pass1_chunk(rows0_v, typ_v, nw_v, sums_v, yv_v)

        @pl.when(g > 0)
        def _():
            wait_scatter(rows1_v, ss1)

        gather(c0 + 1, rows1_v, sg1)
        _pass2_chunk(rows0_v, nw_v, yv_v)
        scatter(c0, rows0_v, ss0)
        wait_gather(rows1_v, sg1)
        _pass1_chunk(rows1_v, typ_v, nw_v, sums_v, yv_v)
        wait_scatter(rows0_v, ss0)

        @pl.when(g < npair - 1)
        def _():
            gather(c0 + 2, rows0_v, sg0)

        _pass2_chunk(rows1_v, nw_v, yv_v)
        scatter(c0 + 1, rows1_v, ss1)
        return carry

    lax.fori_loop(0, npair, body, 0)
    wait_scatter(rows1_v, ss1)


def kernel(input_ids, token_type_ids, word_emb, type_emb, norm_w):
    del token_type_ids
    b, s = input_ids.shape
    n = b * s
    tpw = n // NW
    ids = input_ids.reshape(n)
    typ_row = type_emb.reshape(type_emb.shape[-1])

    mesh = plsc.VectorSubcoreMesh(
        core_axis_name="c", subcore_axis_name="s",
        num_cores=NC, num_subcores=NS)

    k = pl.kernel(
        functools.partial(_sc_body, tpw=tpw),
        out_type=jax.ShapeDtypeStruct((n, D), jnp.float32),
        mesh=mesh,
        compiler_params=pltpu.CompilerParams(needs_layout_passes=False),
        scratch_types=[
            pltpu.VMEM((tpw,), jnp.int32),
            pltpu.VMEM((CH, D), jnp.float32),
            pltpu.VMEM((CH, D), jnp.float32),
            pltpu.VMEM((D,), jnp.float32),
            pltpu.VMEM((D,), jnp.float32),
            pltpu.VMEM((CH * L,), jnp.float32),
            pltpu.VMEM((CH,), jnp.float32),
            pltpu.SemaphoreType.DMA,
            pltpu.SemaphoreType.DMA,
            pltpu.SemaphoreType.DMA,
            pltpu.SemaphoreType.DMA,
        ],
    )
    out = k(ids, word_emb, typ_row, norm_w)
    return out.reshape(b, s, D)


# speedup vs baseline: 1.1927x; 1.0009x over previous
import functools

import jax
import jax.numpy as jnp
from jax import lax
from jax.experimental import pallas as pl
from jax.experimental.pallas import tpu as pltpu
from jax.experimental.pallas import tpu_sc as plsc

D = 1024
L = 16
NVEC = D // L
NC, NS = 2, 16
NW = NC * NS
EPS = 1e-5
CH = 32


JBLK = 16
NBLK = NVEC // JBLK


def _pass1_chunk(rows_v, typ_v, nw_v, sums_v, yv_v):
    zero = jnp.zeros((L,), jnp.float32)

    for j0 in range(NBLK):
        t = [typ_v[pl.ds((j0 * JBLK + k) * L, L)] for k in range(JBLK)]

        @plsc.parallel_loop(0, CH)
        def row_p1(rr, j0=j0, t=t):
            accs = [zero, zero, zero, zero]
            for k in range(JBLK):
                sl = pl.ds((j0 * JBLK + k) * L, L)
                v = rows_v[rr, sl] + t[k]
                rows_v[rr, sl] = v
                accs[k % 4] = accs[k % 4] + v * v
            acc = (accs[0] + accs[1]) + (accs[2] + accs[3])
            if j0 == 0:
                sums_v[pl.ds(rr * L, L)] = acc
            else:
                plsc.addupdate(sums_v.at[pl.ds(rr * L, L)], acc)

    for bb in range(CH // L):
        col = lax.iota(jnp.int32, L) * L + (bb * L * L)

        def tr(cc, a, col=col):
            return a + plsc.load_gather(sums_v, [col + cc])

        acc2 = lax.fori_loop(0, L, tr, zero, unroll=True)
        var = acc2 * (1.0 / D) + EPS
        i = lax.bitcast_convert_type(var, jnp.int32)
        i = jnp.full((L,), 0x5F3759DF, jnp.int32) - \
            lax.shift_right_arithmetic(i, jnp.full((L,), 1, jnp.int32))
        y = lax.bitcast_convert_type(i, jnp.float32)
        h = 0.5 * var
        y = y * (1.5 - h * y * y)
        y = y * (1.5 - h * y * y)
        y = y * (1.5 - h * y * y)
        yv_v[pl.ds(bb * L, L)] = y


JBLK2 = 32


def _pass2_chunk(rows_v, nw_v, yv_v):
    for j0 in range(NVEC // JBLK2):
        w = [nw_v[pl.ds((j0 * JBLK2 + k) * L, L)] for k in range(JBLK2)]

        @plsc.parallel_loop(0, CH)
        def row_p2(rr, j0=j0, w=w):
            s = plsc.load_gather(yv_v, [rr + jnp.zeros((L,), jnp.int32)])
            for k in range(JBLK2):
                sl = pl.ds((j0 * JBLK2 + k) * L, L)
                rows_v[rr, sl] = rows_v[rr, sl] * s * w[k]


def _sc_body(ids_hbm, table_hbm, typ_hbm, nw_hbm, out_hbm,
             idx_v, rows0_v, rows1_v, typ_v, nw_v, sums_v, yv_v,
             sg0, sg1, ss0, ss1, tpw):
    wid = lax.axis_index("s") * NC + lax.axis_index("c")
    base = wid * tpw

    nch = tpw // CH
    npair = nch // 2

    def gather(c, buf, sem):
        pltpu.async_copy(table_hbm.at[idx_v.at[pl.ds(c * CH, CH)]], buf, sem)

    def wait_gather(buf, sem):
        pltpu.make_async_copy(
            table_hbm.at[idx_v.at[pl.ds(0, CH)]], buf, sem).wait()

    def scatter(c, buf, sem):
        pltpu.async_copy(buf, out_hbm.at[pl.ds(base + c * CH, CH)], sem)

    def wait_scatter(buf, sem):
        pltpu.make_async_copy(buf, out_hbm.at[pl.ds(base, CH)], sem).wait()

    pltpu.sync_copy(ids_hbm.at[pl.ds(base, CH)], idx_v.at[pl.ds(0, CH)])
    gather(0, rows0_v, sg0)
    pltpu.sync_copy(ids_hbm.at[pl.ds(base + CH, tpw - CH)],
                    idx_v.at[pl.ds(CH, tpw - CH)])
    pltpu.sync_copy(typ_hbm, typ_v)
    pltpu.sync_copy(nw_hbm, nw_v)

    def body(g, carry):
        c0 = 2 * g
        wait_gather(rows0_v, sg0)
        _pass1_chunk(rows0_v, typ_v, nw_v, sums_v, yv_v)

        @pl.when(g > 0)
        def _():
            wait_scatter(rows1_v, ss1)

        gather(c0 + 1, rows1_v, sg1)
        _pass2_chunk(rows0_v, nw_v, yv_v)
        scatter(c0, rows0_v, ss0)
        wait_gather(rows1_v, sg1)
        _pass1_chunk(rows1_v, typ_v, nw_v, sums_v, yv_v)
        wait_scatter(rows0_v, ss0)

        @pl.when(g < npair - 1)
        def _():
            gather(c0 + 2, rows0_v, sg0)

        _pass2_chunk(rows1_v, nw_v, yv_v)
        scatter(c0 + 1, rows1_v, ss1)
        return carry

    lax.fori_loop(0, npair, body, 0)
    wait_scatter(rows1_v, ss1)


def kernel(input_ids, token_type_ids, word_emb, type_emb, norm_w):
    del token_type_ids
    b, s = input_ids.shape
    n = b * s
    tpw = n // NW
    ids = input_ids.reshape(n)
    typ_row = type_emb.reshape(type_emb.shape[-1])

    mesh = plsc.VectorSubcoreMesh(
        core_axis_name="c", subcore_axis_name="s",
        num_cores=NC, num_subcores=NS)

    k = pl.kernel(
        functools.partial(_sc_body, tpw=tpw),
        out_type=jax.ShapeDtypeStruct((n, D), jnp.float32),
        mesh=mesh,
        compiler_params=pltpu.CompilerParams(needs_layout_passes=False),
        scratch_types=[
            pltpu.VMEM((tpw,), jnp.int32),
            pltpu.VMEM((CH, D), jnp.float32),
            pltpu.VMEM((CH, D), jnp.float32),
            pltpu.VMEM((D,), jnp.float32),
            pltpu.VMEM((D,), jnp.float32),
            pltpu.VMEM((CH * L,), jnp.float32),
            pltpu.VMEM((CH,), jnp.float32),
            pltpu.SemaphoreType.DMA,
            pltpu.SemaphoreType.DMA,
            pltpu.SemaphoreType.DMA,
            pltpu.SemaphoreType.DMA,
        ],
    )
    out = k(ids, word_emb, typ_row, norm_w)
    return out.reshape(b, s, D)
